# baseline (speedup 1.0000x reference)
import jax
import jax.numpy as jnp
from jax import lax
from jax.experimental import pallas as pl
from jax.experimental.pallas import tpu as pltpu
from jax.experimental.pallas import tpu_sc as plsc

N = 50000
E = 800000
H = 96
MSG = 64
HID = 128
DS = 16
DE = 16
DDYN = 8

NW = 32
EPAD = 819200
PER_W = EPAD // NW
BE = 64
NB1 = PER_W // BE
NJ1 = BE // 16
PER_T2 = EPAD // 16
BE2 = 64
NB2 = PER_T2 // BE2
NROW_T = 3128
NPADM = NROW_T * 16

_f32 = jnp.float32
_i32 = jnp.int32


def _a0_body(wes2_ref, wbw_ref, bes2_ref, bbw_ref, bg2_ref, gc_ref):
    w2row = lax.dot_general(
        wbw_ref[...], wes2_ref[...],
        dimension_numbers=(((0,), (1,)), ((), ())),
        preferred_element_type=_f32)
    c2 = jnp.dot(bes2_ref[...], wbw_ref[...],
                 preferred_element_type=_f32) + bbw_ref[...]
    io = lax.broadcasted_iota(_i32, (1, 128), 1)
    row1 = jnp.where(io == 0, c2[0, 0], 0.0) + jnp.where(io == 1, bg2_ref[0, 0], 0.0)
    gc_ref[0:1, :] = w2row
    gc_ref[1:2, :] = row1
    gc_ref[2:8, :] = jnp.zeros((6, 128), _f32)


def _a1_body(h_ref, xs_ref, wg1a_ref, wg1b_ref, bg1_ref, ws_ref, wd_ref,
             wp1_ref, bp1_ref, wp2_ref, bp2_ref,
             srct_ref, dstt_ref, v_ref):
    h = h_ref[...]
    xs = xs_ref[...]
    bf = jnp.bfloat16
    srct_ref[:, 0:128] = jnp.dot(
        xs, ws_ref[...], preferred_element_type=_f32).astype(bf)
    srct_ref[:, 128:256] = (jnp.dot(h, wg1a_ref[...], preferred_element_type=_f32)
                            + bg1_ref[...]).astype(bf)
    dstt_ref[:, 0:128] = jnp.dot(
        xs, wd_ref[...], preferred_element_type=_f32).astype(bf)
    dstt_ref[:, 128:256] = jnp.dot(
        h, wg1b_ref[...], preferred_element_type=_f32).astype(bf)
    hp = jnp.maximum(jnp.dot(h, wp1_ref[...], preferred_element_type=_f32)
                     + bp1_ref[...], 0.0)
    v_ref[:, 0:MSG] = (jnp.dot(hp, wp2_ref[...], preferred_element_type=_f32)
                       + bp2_ref[...]).astype(jnp.bfloat16)
    v_ref[:, MSG:128] = jnp.zeros((h.shape[0], 128 - MSG), jnp.bfloat16)


def _a2_body(ea_ref, wea_ref, bes1_ref, out_ref):
    out_ref[...] = (jnp.dot(ea_ref[...], wea_ref[...], preferred_element_type=_f32)
                    + bes1_ref[...]).astype(jnp.bfloat16)


def _softplus16(x):
    a = jnp.abs(x)
    t = jnp.exp(-a)
    u = t / (2.0 + t)
    u2 = u * u
    at = u * (1.0 + u2 * (1.0 / 3.0 + u2 * (0.2 + u2 * (1.0 / 7.0 + u2 * (1.0 / 9.0)))))
    return jnp.maximum(x, 0.0) + 2.0 * at


def _p1_body(sidx_hbm, didx_hbm, srct_hbm, dstt_hbm, ea_hbm, gc_hbm,
             w2p_hbm, wg2p_hbm,
             c_hbm,
             sidx0, sidx1, sidx2, sidx3, didx0, didx1, didx2, didx3,
             srows0, srows1, srows2, srows3, drows0, drows1, drows2, drows3,
             ea0, ea1, ea2, ea3, cb0, cb1, w2v, wg2v, ccv, tb1, tb2,
             semi0, semi1, semi2, semi3, semg0, semg1, semg2, semg3,
             semo0, semo1):
    cid = lax.axis_index("c")
    sid = lax.axis_index("s")
    wid = sid * 2 + cid
    pltpu.sync_copy(gc_hbm.at[1], ccv)
    pltpu.sync_copy(w2p_hbm, w2v)
    pltpu.sync_copy(wg2p_hbm, wg2v)
    base0 = wid * PER_W
    sx = (sidx0, sidx1, sidx2, sidx3)
    dx = (didx0, didx1, didx2, didx3)
    sr = (srows0, srows1, srows2, srows3)
    dr = (drows0, drows1, drows2, drows3)
    eab = (ea0, ea1, ea2, ea3)
    cbs = (cb0, cb1)
    semi = (semi0, semi1, semi2, semi3)
    semg = (semg0, semg1, semg2, semg3)
    semo = (semo0, semo1)
    w2c = [w2v[pl.ds(16 * k, 16)] for k in range(8)]
    wg2c = [wg2v[pl.ds(16 * k, 16)] for k in range(8)]
    cc16 = ccv[0:16]
    c2 = cc16[0]
    bg2 = cc16[1]
    iota = lax.iota(_i32, 16)
    tcols = [iota * 17 + f for f in range(16)]

    def ibase(b):
        return base0 + b * BE

    def issue_idx(b, s):
        pltpu.async_copy(sidx_hbm.at[pl.ds(ibase(b), BE)], sx[s], semi[s])
        pltpu.async_copy(didx_hbm.at[pl.ds(ibase(b), BE)], dx[s], semi[s])

    def wait_idx(b, s):
        pltpu.make_async_copy(sidx_hbm.at[pl.ds(ibase(b), BE)], sx[s], semi[s]).wait()
        pltpu.make_async_copy(didx_hbm.at[pl.ds(ibase(b), BE)], dx[s], semi[s]).wait()

    def issue_g(b, s):
        pltpu.async_copy(srct_hbm.at[sx[s]], sr[s], semg[s])
        pltpu.async_copy(dstt_hbm.at[dx[s]], dr[s], semg[s])
        pltpu.async_copy(ea_hbm.at[pl.ds(ibase(b), BE)], eab[s], semg[s])

    def wait_g(b, s):
        pltpu.make_async_copy(srct_hbm.at[sx[s]], sr[s], semg[s]).wait()
        pltpu.make_async_copy(dstt_hbm.at[dx[s]], dr[s], semg[s]).wait()
        pltpu.make_async_copy(ea_hbm.at[pl.ds(ibase(b), BE)], eab[s], semg[s]).wait()

    IL = plsc.PackFormat.INTERLEAVED

    def compute_batch(base, s, p):
        srows = sr[s]
        drows = dr[s]
        earows = eab[s]
        cbuf = cbs[p]

        def jgroup(j, carry2):
            for i in range(16):
                e = j * 16 + i
                p1 = None
                p2 = None
                for k in range(4):
                    sl = pl.ds(32 * k, 32)
                    slg = pl.ds(128 + 32 * k, 32)
                    ea_u = plsc.unpack(earows[e, sl], format=IL)
                    s_u = plsc.unpack(srows[e, sl], format=IL)
                    d_u = plsc.unpack(drows[e, sl], format=IL)
                    a_u = plsc.unpack(srows[e, slg], format=IL)
                    b_u = plsc.unpack(drows[e, slg], format=IL)
                    for hh in range(2):
                        r1 = jnp.maximum(ea_u[hh] + s_u[hh] + d_u[hh], 0.0)
                        r2 = jnp.maximum(a_u[hh] + b_u[hh], 0.0)
                        q1 = r1 * w2c[2 * k + hh]
                        q2 = r2 * wg2c[2 * k + hh]
                        p1 = q1 if p1 is None else p1 + q1
                        p2 = q2 if p2 is None else p2 + q2
                tb1[pl.ds(i * 17, 16)] = p1
                tb2[pl.ds(i * 17, 16)] = p2
            u1 = None
            u2 = None
            for f in range(16):
                g1 = plsc.load_gather(tb1, [tcols[f]])
                g2 = plsc.load_gather(tb2, [tcols[f]])
                u1 = g1 if u1 is None else u1 + g1
                u2 = g2 if u2 is None else u2 + g2
            be = _softplus16(u1 + c2)
            ge = 1.0 / (1.0 + jnp.exp(-(u2 + bg2)))
            cv = be * ge
            eid = iota + (base + 16 * j)
            cv = jnp.where(eid < E, cv, 0.0)
            cbuf[pl.ds(16 * j, 16)] = cv
            return carry2

        lax.fori_loop(0, NJ1, jgroup, 0)

    for s0 in range(4):
        issue_idx(s0, s0)
    for s0 in range(3):
        wait_idx(s0, s0)
        issue_g(s0, s0)

    def body4(t4, carry):
        for pp in range(4):
            t = t4 * 4 + pp
            s = pp
            p = pp % 2
            s3 = (pp + 3) % 4

            @pl.when(t + 3 < NB1)
            def _():
                wait_idx(t + 3, s3)
                issue_g(t + 3, s3)

            wait_g(t, s)

            @pl.when(t >= 2)
            def _():
                pltpu.make_async_copy(
                    cbs[p], c_hbm.at[pl.ds(ibase(t - 2), BE)], semo[p]).wait()

            compute_batch(ibase(t), s, p)
            pltpu.async_copy(cbs[p], c_hbm.at[pl.ds(ibase(t), BE)], semo[p])

            @pl.when(t + 4 < NB1)
            def _():
                issue_idx(t + 4, s)
        return carry

    lax.fori_loop(0, NB1 // 4, body4, 0)
    pltpu.make_async_copy(cbs[0], c_hbm.at[pl.ds(ibase(NB1 - 2), BE)], semo[0]).wait()
    pltpu.make_async_copy(cbs[1], c_hbm.at[pl.ds(ibase(NB1 - 1), BE)], semo[1]).wait()


def _p2_body(sidx_hbm, didx_hbm, c_hbm, v_hbm, z_hbm,
             m_hbm,
             sidx0, sidx1, sidx2, sidx3, didx0, didx1, didx2, didx3,
             cb0, cb1, cb2, cb3, dsc0, dsc1,
             vr0, vr1, vr2, vr3, mg0, mg1, mspm,
             semi0, semi1, semi2, semi3, semg0, semg1, semg2, semg3,
             sems0, sems1):
    cid = lax.axis_index("c")
    sid = lax.axis_index("s")
    pltpu.sync_copy(z_hbm, mspm.at[pl.ds(sid * NROW_T, NROW_T)])
    plsc.subcore_barrier()

    base0 = sid * PER_T2
    off = cid * 32
    sx = (sidx0, sidx1, sidx2, sidx3)
    dx = (didx0, didx1, didx2, didx3)
    cbs = (cb0, cb1, cb2, cb3)
    dsc = (dsc0, dsc1)
    vr = (vr0, vr1, vr2, vr3)
    mg = (mg0, mg1)
    semi = (semi0, semi1, semi2, semi3)
    semg = (semg0, semg1, semg2, semg3)
    sems = (sems0, sems1)

    def ibase(b):
        return base0 + b * BE2

    def issue_idx(b, s):
        pltpu.async_copy(sidx_hbm.at[pl.ds(ibase(b), BE2)], sx[s], semi[s])
        pltpu.async_copy(didx_hbm.at[pl.ds(ibase(b), BE2)], dx[s], semi[s])
        pltpu.async_copy(c_hbm.at[pl.ds(ibase(b), BE2)], cbs[s], semi[s])

    def wait_idx(b, s):
        pltpu.make_async_copy(sidx_hbm.at[pl.ds(ibase(b), BE2)], sx[s], semi[s]).wait()
        pltpu.make_async_copy(didx_hbm.at[pl.ds(ibase(b), BE2)], dx[s], semi[s]).wait()
        pltpu.make_async_copy(c_hbm.at[pl.ds(ibase(b), BE2)], cbs[s], semi[s]).wait()

    def issue_g(s):
        pltpu.async_copy(v_hbm.at[sx[s]], vr[s], semg[s])

    def wait_g(s):
        pltpu.make_async_copy(v_hbm.at[sx[s]], vr[s], semg[s]).wait()

    for s0 in range(4):
        issue_idx(s0, s0)
    for s0 in range(3):
        wait_idx(s0, s0)
        issue_g(s0)

    def body4(t4, carry):
        for pp in range(4):
            t = t4 * 4 + pp
            s = pp
            p = pp % 2
            s3 = (pp + 3) % 4

            @pl.when(t + 3 < NB2)
            def _():
                wait_idx(t + 3, s3)
                issue_g(s3)

            wait_g(s)

            @pl.when(t >= 2)
            def _():
                pltpu.make_async_copy(mg[p], mspm.at[dsc[p]], sems[p]).wait()

            vrows = vr[s]
            msgb = mg[p]
            cb = cbs[s]
            dst_s = dsc[p]
            for i4 in range(BE2 // 16):
                dst_s[pl.ds(i4 * 16, 16)] = dx[s][pl.ds(i4 * 16, 16)]

            def jgroup(j, carry2):
                cv = cb[pl.ds(j * 16, 16)]
                for i in range(16):
                    e = j * 16 + i
                    ce = cv[i]
                    va, vb = plsc.unpack(vrows[e, pl.ds(off, 32)],
                                         format=plsc.PackFormat.INTERLEAVED)
                    msgb[e, 0:16] = ce * va
                    msgb[e, 16:32] = ce * vb
                return carry2

            lax.fori_loop(0, BE2 // 16, jgroup, 0)
            pltpu.async_copy(msgb, mspm.at[dst_s], sems[p], add=True)

            @pl.when(t + 4 < NB2)
            def _():
                issue_idx(t + 4, s)
        return carry

    lax.fori_loop(0, NB2 // 4, body4, 0)
    pltpu.make_async_copy(mg[0], mspm.at[dsc[0]], sems[0]).wait()
    pltpu.make_async_copy(mg[1], mspm.at[dsc[1]], sems[1]).wait()
    plsc.subcore_barrier()
    pltpu.sync_copy(mspm.at[pl.ds(sid * NROW_T, NROW_T)],
                    m_hbm.at[cid, pl.ds(sid * NROW_T, NROW_T)])


def _c_body(m_ref, xd_ref, h_ref, wdyn_ref, bdyn_ref,
            wir_m_ref, wiz_m_ref, win_m_ref, wir_d_ref, wiz_d_ref, win_d_ref,
            bir_ref, biz_ref, bin_ref,
            whr_ref, whz_ref, whn_ref, bhr_ref, bhz_ref, bhn_ref,
            gm_ref, btm_ref, gd_ref, btd_ref, gh_ref, bth_ref,
            out_ref):
    def ln(x, g, b):
        mu = jnp.mean(x, axis=-1, keepdims=True)
        var = jnp.mean((x - mu) ** 2, axis=-1, keepdims=True)
        return (x - mu) * jax.lax.rsqrt(var + 1e-5) * g + b

    m = ln(m_ref[...], gm_ref[...], btm_ref[...])
    d = ln(jnp.dot(xd_ref[...], wdyn_ref[...], preferred_element_type=_f32)
           + bdyn_ref[...], gd_ref[...], btd_ref[...])
    h = h_ref[...]

    def mm(x, w):
        return jnp.dot(x, w[...], preferred_element_type=_f32)

    i_r = mm(m, wir_m_ref) + mm(d, wir_d_ref) + bir_ref[...]
    i_z = mm(m, wiz_m_ref) + mm(d, wiz_d_ref) + biz_ref[...]
    i_n = mm(m, win_m_ref) + mm(d, win_d_ref) + bin_ref[...]
    h_r = mm(h, whr_ref) + bhr_ref[...]
    h_z = mm(h, whz_ref) + bhz_ref[...]
    h_n = mm(h, whn_ref) + bhn_ref[...]
    r = jax.nn.sigmoid(i_r + h_r)
    z = jax.nn.sigmoid(i_z + h_z)
    n = jnp.tanh(i_n + r * h_n)
    h_new = (1.0 - z) * n + z * h
    out_ref[...] = ln(h_new, gh_ref[...], bth_ref[...])


def kernel(h, x_static, x_dyn, edge_attr_static, edge_index,
           W_es1, b_es1, W_es2, b_es2, W_bw, b_bw, W_g1, b_g1, W_g2, b_g2,
           W_p1, b_p1, W_p2, b_p2, W_dyn, b_dyn, W_ih, b_ih, W_hh, b_hh,
           g_msg, bt_msg, g_dyn, bt_dyn, g_h, bt_h):
    r1 = lambda a: a.reshape(1, -1)

    gc = pl.pallas_call(
        _a0_body,
        out_shape=jax.ShapeDtypeStruct((8, 128), _f32),
    )(W_es2, W_bw, r1(b_es2), r1(b_bw), r1(b_g2))

    BN = 2000
    gridn = N // BN
    full = lambda shp: pl.BlockSpec(shp, lambda i: (0, 0))
    srct, dstt, vtab = pl.pallas_call(
        _a1_body,
        grid=(gridn,),
        in_specs=[
            pl.BlockSpec((BN, H), lambda i: (i, 0)),
            pl.BlockSpec((BN, DS), lambda i: (i, 0)),
            full((H, HID)), full((H, HID)), full((1, HID)),
            full((DS, HID)), full((DS, HID)),
            full((H, HID)), full((1, HID)), full((HID, MSG)), full((1, MSG)),
        ],
        out_specs=[
            pl.BlockSpec((BN, 256), lambda i: (i, 0)),
            pl.BlockSpec((BN, 256), lambda i: (i, 0)),
            pl.BlockSpec((BN, 128), lambda i: (i, 0)),
        ],
        out_shape=[
            jax.ShapeDtypeStruct((N, 256), jnp.bfloat16),
            jax.ShapeDtypeStruct((N, 256), jnp.bfloat16),
            jax.ShapeDtypeStruct((N, 128), jnp.bfloat16),
        ],
    )(h, x_static, W_g1[:H], W_g1[H:], r1(b_g1), W_es1[DE:DE + DS],
      W_es1[DE + DS:], W_p1, r1(b_p1), W_p2, r1(b_p2))

    ea_pad = jnp.pad(edge_attr_static, ((0, EPAD - E), (0, 0)))
    BEA = 8192
    ea_proj = pl.pallas_call(
        _a2_body,
        grid=(EPAD // BEA,),
        in_specs=[
            pl.BlockSpec((BEA, DE), lambda i: (i, 0)),
            full((DE, HID)), full((1, HID)),
        ],
        out_specs=pl.BlockSpec((BEA, HID), lambda i: (i, 0)),
        out_shape=jax.ShapeDtypeStruct((EPAD, HID), jnp.bfloat16),
    )(ea_pad, W_es1[:DE], r1(b_es1))

    src = jnp.pad(edge_index[0], (0, EPAD - E))
    dst = jnp.pad(edge_index[1], (0, EPAD - E))

    mesh = plsc.VectorSubcoreMesh(core_axis_name="c", subcore_axis_name="s")
    scp = pltpu.CompilerParams(needs_layout_passes=False,
                               use_tc_tiling_on_sc=False)
    perm = lambda w: w.reshape(4, 16, 2).transpose(0, 2, 1).reshape(128)
    w2p = perm(gc[0])
    wg2p = perm(W_g2[:, 0])
    bf = jnp.bfloat16
    c = pl.kernel(
        _p1_body,
        out_type=jax.ShapeDtypeStruct((EPAD,), _f32),
        mesh=mesh,
        compiler_params=scp,
        scratch_types=(
            [pltpu.VMEM((BE,), _i32) for _ in range(8)]
            + [pltpu.VMEM((BE, 256), bf) for _ in range(8)]
            + [pltpu.VMEM((BE, 128), bf) for _ in range(4)]
            + [pltpu.VMEM((BE,), _f32) for _ in range(2)]
            + [pltpu.VMEM((128,), _f32) for _ in range(3)]
            + [pltpu.VMEM((272,), _f32) for _ in range(2)]
            + [pltpu.SemaphoreType.DMA for _ in range(10)]
        ),
    )(src, dst, srct, dstt, ea_proj, gc, w2p, wg2p)

    zrows = jnp.zeros((NROW_T, 32), _f32)
    m2 = pl.kernel(
        _p2_body,
        out_type=jax.ShapeDtypeStruct((2, NPADM, 32), _f32),
        mesh=mesh,
        compiler_params=scp,
        scratch_types=(
            [pltpu.VMEM((BE2,), _i32) for _ in range(8)]
            + [pltpu.VMEM((BE2,), _f32) for _ in range(4)]
            + [pltpu.VMEM((BE2,), _i32) for _ in range(2)]
            + [pltpu.VMEM((BE2, 128), jnp.bfloat16) for _ in range(4)]
            + [pltpu.VMEM((BE2, 32), _f32) for _ in range(2)]
            + [pltpu.VMEM_SHARED((NPADM, 32), _f32)]
            + [pltpu.SemaphoreType.DMA for _ in range(10)]
        ),
    )(src, dst, c, vtab, zrows)

    inv = [(f // 2 if f % 2 == 0 else 16 + f // 2) for f in range(32)]
    m = jnp.concatenate([m2[0, :N][:, inv], m2[1, :N][:, inv]], axis=1)

    Wim, Wid = W_ih[:MSG], W_ih[MSG:]
    out = pl.pallas_call(
        _c_body,
        grid=(gridn,),
        in_specs=[
            pl.BlockSpec((BN, MSG), lambda i: (i, 0)),
            pl.BlockSpec((BN, DDYN), lambda i: (i, 0)),
            pl.BlockSpec((BN, H), lambda i: (i, 0)),
            full((DDYN, MSG)), full((1, MSG)),
            full((MSG, H)), full((MSG, H)), full((MSG, H)),
            full((MSG, H)), full((MSG, H)), full((MSG, H)),
            full((1, H)), full((1, H)), full((1, H)),
            full((H, H)), full((H, H)), full((H, H)),
            full((1, H)), full((1, H)), full((1, H)),
            full((1, MSG)), full((1, MSG)), full((1, MSG)), full((1, MSG)),
            full((1, H)), full((1, H)),
        ],
        out_specs=pl.BlockSpec((BN, H), lambda i: (i, 0)),
        out_shape=jax.ShapeDtypeStruct((N, H), _f32),
    )(m, x_dyn, h, W_dyn, r1(b_dyn),
      Wim[:, :H], Wim[:, H:2 * H], Wim[:, 2 * H:],
      Wid[:, :H], Wid[:, H:2 * H], Wid[:, 2 * H:],
      r1(b_ih[:H]), r1(b_ih[H:2 * H]), r1(b_ih[2 * H:]),
      W_hh[:, :H], W_hh[:, H:2 * H], W_hh[:, 2 * H:],
      r1(b_hh[:H]), r1(b_hh[H:2 * H]), r1(b_hh[2 * H:]),
      r1(g_msg), r1(bt_msg), r1(g_dyn), r1(bt_dyn),
      r1(g_h), r1(bt_h))
    return out


# speedup vs baseline: 1.0244x; 1.0244x over previous
import jax
import jax.numpy as jnp
from jax import lax
from jax.experimental import pallas as pl
from jax.experimental.pallas import tpu as pltpu
from jax.experimental.pallas import tpu_sc as plsc

N = 50000
E = 800000
H = 96
MSG = 64
HID = 128
DS = 16
DE = 16
DDYN = 8

NW = 32
EPAD = 819200
PER_W = EPAD // NW
BE = 32
NB1 = PER_W // BE
NJ1 = BE // 16
PER_T2 = EPAD // 16
BE2 = 64
NB2 = PER_T2 // BE2
NROW_T = 3128
NPADM = NROW_T * 16

_f32 = jnp.float32
_i32 = jnp.int32


def _a0_body(wes2_ref, wbw_ref, bes2_ref, bbw_ref, bg2_ref, gc_ref):
    w2row = lax.dot_general(
        wbw_ref[...], wes2_ref[...],
        dimension_numbers=(((0,), (1,)), ((), ())),
        preferred_element_type=_f32)
    c2 = jnp.dot(bes2_ref[...], wbw_ref[...],
                 preferred_element_type=_f32) + bbw_ref[...]
    io = lax.broadcasted_iota(_i32, (1, 128), 1)
    row1 = jnp.where(io == 0, c2[0, 0], 0.0) + jnp.where(io == 1, bg2_ref[0, 0], 0.0)
    gc_ref[0:1, :] = w2row
    gc_ref[1:2, :] = row1
    gc_ref[2:8, :] = jnp.zeros((6, 128), _f32)


def _a1_body(h_ref, xs_ref, wg1a_ref, wg1b_ref, bg1_ref, ws_ref, wd_ref,
             wp1_ref, bp1_ref, wp2_ref, bp2_ref,
             srct_ref, dstt_ref, v_ref):
    h = h_ref[...]
    xs = xs_ref[...]
    srct_ref[:, 0:128] = jnp.dot(xs, ws_ref[...], preferred_element_type=_f32)
    srct_ref[:, 128:256] = (jnp.dot(h, wg1a_ref[...], preferred_element_type=_f32)
                            + bg1_ref[...])
    dstt_ref[:, 0:128] = jnp.dot(xs, wd_ref[...], preferred_element_type=_f32)
    dstt_ref[:, 128:256] = jnp.dot(h, wg1b_ref[...], preferred_element_type=_f32)
    hp = jnp.maximum(jnp.dot(h, wp1_ref[...], preferred_element_type=_f32)
                     + bp1_ref[...], 0.0)
    v_ref[:, 0:MSG] = (jnp.dot(hp, wp2_ref[...], preferred_element_type=_f32)
                       + bp2_ref[...]).astype(jnp.bfloat16)
    v_ref[:, MSG:128] = jnp.zeros((h.shape[0], 128 - MSG), jnp.bfloat16)


def _a2_body(ea_ref, wea_ref, bes1_ref, out_ref):
    out_ref[...] = (jnp.dot(ea_ref[...], wea_ref[...], preferred_element_type=_f32)
                    + bes1_ref[...])


def _softplus16(x):
    a = jnp.abs(x)
    t = jnp.exp(-a)
    u = t / (2.0 + t)
    u2 = u * u
    at = u * (1.0 + u2 * (1.0 / 3.0 + u2 * (0.2 + u2 * (1.0 / 7.0 + u2 * (1.0 / 9.0)))))
    return jnp.maximum(x, 0.0) + 2.0 * at


def _p1_body(sidx_hbm, didx_hbm, srct_hbm, dstt_hbm, ea_hbm, gc_hbm,
             w2p_hbm, wg2p_hbm,
             c_hbm,
             sidx0, sidx1, sidx2, sidx3, didx0, didx1, didx2, didx3,
             srows0, srows1, srows2, srows3, drows0, drows1, drows2, drows3,
             ea0, ea1, ea2, ea3, cb0, cb1, w2v, wg2v, ccv, tb1, tb2,
             semi0, semi1, semi2, semi3, semg0, semg1, semg2, semg3,
             semo0, semo1):
    cid = lax.axis_index("c")
    sid = lax.axis_index("s")
    wid = sid * 2 + cid
    pltpu.sync_copy(gc_hbm.at[1], ccv)
    pltpu.sync_copy(w2p_hbm, w2v)
    pltpu.sync_copy(wg2p_hbm, wg2v)
    base0 = wid * PER_W
    sx = (sidx0, sidx1, sidx2, sidx3)
    dx = (didx0, didx1, didx2, didx3)
    sr = (srows0, srows1, srows2, srows3)
    dr = (drows0, drows1, drows2, drows3)
    eab = (ea0, ea1, ea2, ea3)
    cbs = (cb0, cb1)
    semi = (semi0, semi1, semi2, semi3)
    semg = (semg0, semg1, semg2, semg3)
    semo = (semo0, semo1)
    w2c = [w2v[pl.ds(16 * k, 16)] for k in range(8)]
    wg2c = [wg2v[pl.ds(16 * k, 16)] for k in range(8)]
    cc16 = ccv[0:16]
    c2 = cc16[0]
    bg2 = cc16[1]
    iota = lax.iota(_i32, 16)
    tcols = [iota * 17 + f for f in range(16)]

    def ibase(b):
        return base0 + b * BE

    def issue_idx(b, s):
        pltpu.async_copy(sidx_hbm.at[pl.ds(ibase(b), BE)], sx[s], semi[s])
        pltpu.async_copy(didx_hbm.at[pl.ds(ibase(b), BE)], dx[s], semi[s])

    def wait_idx(b, s):
        pltpu.make_async_copy(sidx_hbm.at[pl.ds(ibase(b), BE)], sx[s], semi[s]).wait()
        pltpu.make_async_copy(didx_hbm.at[pl.ds(ibase(b), BE)], dx[s], semi[s]).wait()

    def issue_g(b, s):
        pltpu.async_copy(srct_hbm.at[sx[s]], sr[s], semg[s])
        pltpu.async_copy(dstt_hbm.at[dx[s]], dr[s], semg[s])
        pltpu.async_copy(ea_hbm.at[pl.ds(ibase(b), BE)], eab[s], semg[s])

    def wait_g(b, s):
        pltpu.make_async_copy(srct_hbm.at[sx[s]], sr[s], semg[s]).wait()
        pltpu.make_async_copy(dstt_hbm.at[dx[s]], dr[s], semg[s]).wait()
        pltpu.make_async_copy(ea_hbm.at[pl.ds(ibase(b), BE)], eab[s], semg[s]).wait()

    IL = plsc.PackFormat.INTERLEAVED

    def compute_batch(base, s, p):
        srows = sr[s]
        drows = dr[s]
        earows = eab[s]
        cbuf = cbs[p]

        def jgroup(j, carry2):
            for i in range(16):
                e = j * 16 + i
                p1 = None
                p2 = None
                for k in range(8):
                    sl = pl.ds(16 * k, 16)
                    slg = pl.ds(128 + 16 * k, 16)
                    r1 = jnp.maximum(earows[e, sl] + srows[e, sl] + drows[e, sl], 0.0)
                    r2 = jnp.maximum(srows[e, slg] + drows[e, slg], 0.0)
                    q1 = r1 * w2c[k]
                    q2 = r2 * wg2c[k]
                    p1 = q1 if p1 is None else p1 + q1
                    p2 = q2 if p2 is None else p2 + q2
                tb1[pl.ds(i * 17, 16)] = p1
                tb2[pl.ds(i * 17, 16)] = p2
            u1 = None
            u2 = None
            for f in range(16):
                g1 = plsc.load_gather(tb1, [tcols[f]])
                g2 = plsc.load_gather(tb2, [tcols[f]])
                u1 = g1 if u1 is None else u1 + g1
                u2 = g2 if u2 is None else u2 + g2
            be = _softplus16(u1 + c2)
            ge = 1.0 / (1.0 + jnp.exp(-(u2 + bg2)))
            cv = be * ge
            eid = iota + (base + 16 * j)
            cv = jnp.where(eid < E, cv, 0.0)
            cbuf[pl.ds(16 * j, 16)] = cv
            return carry2

        lax.fori_loop(0, NJ1, jgroup, 0)

    for s0 in range(4):
        issue_idx(s0, s0)
    for s0 in range(3):
        wait_idx(s0, s0)
        issue_g(s0, s0)

    def body4(t4, carry):
        for pp in range(4):
            t = t4 * 4 + pp
            s = pp
            p = pp % 2
            s3 = (pp + 3) % 4

            @pl.when(t + 3 < NB1)
            def _():
                wait_idx(t + 3, s3)
                issue_g(t + 3, s3)

            wait_g(t, s)

            @pl.when(t >= 2)
            def _():
                pltpu.make_async_copy(
                    cbs[p], c_hbm.at[pl.ds(ibase(t - 2), BE)], semo[p]).wait()

            compute_batch(ibase(t), s, p)
            pltpu.async_copy(cbs[p], c_hbm.at[pl.ds(ibase(t), BE)], semo[p])

            @pl.when(t + 4 < NB1)
            def _():
                issue_idx(t + 4, s)
        return carry

    lax.fori_loop(0, NB1 // 4, body4, 0)
    pltpu.make_async_copy(cbs[0], c_hbm.at[pl.ds(ibase(NB1 - 2), BE)], semo[0]).wait()
    pltpu.make_async_copy(cbs[1], c_hbm.at[pl.ds(ibase(NB1 - 1), BE)], semo[1]).wait()


def _p2_body(sidx_hbm, didx_hbm, c_hbm, v_hbm, z_hbm,
             m_hbm,
             sidx0, sidx1, sidx2, sidx3, didx0, didx1, didx2, didx3,
             cb0, cb1, cb2, cb3, dsc0, dsc1,
             vr0, vr1, vr2, vr3, mg0, mg1, mspm,
             semi0, semi1, semi2, semi3, semg0, semg1, semg2, semg3,
             sems0, sems1):
    cid = lax.axis_index("c")
    sid = lax.axis_index("s")
    pltpu.sync_copy(z_hbm, mspm.at[pl.ds(sid * NROW_T, NROW_T)])
    plsc.subcore_barrier()

    base0 = sid * PER_T2
    off = cid * 32
    sx = (sidx0, sidx1, sidx2, sidx3)
    dx = (didx0, didx1, didx2, didx3)
    cbs = (cb0, cb1, cb2, cb3)
    dsc = (dsc0, dsc1)
    vr = (vr0, vr1, vr2, vr3)
    mg = (mg0, mg1)
    semi = (semi0, semi1, semi2, semi3)
    semg = (semg0, semg1, semg2, semg3)
    sems = (sems0, sems1)

    def ibase(b):
        return base0 + b * BE2

    def issue_idx(b, s):
        pltpu.async_copy(sidx_hbm.at[pl.ds(ibase(b), BE2)], sx[s], semi[s])
        pltpu.async_copy(didx_hbm.at[pl.ds(ibase(b), BE2)], dx[s], semi[s])
        pltpu.async_copy(c_hbm.at[pl.ds(ibase(b), BE2)], cbs[s], semi[s])

    def wait_idx(b, s):
        pltpu.make_async_copy(sidx_hbm.at[pl.ds(ibase(b), BE2)], sx[s], semi[s]).wait()
        pltpu.make_async_copy(didx_hbm.at[pl.ds(ibase(b), BE2)], dx[s], semi[s]).wait()
        pltpu.make_async_copy(c_hbm.at[pl.ds(ibase(b), BE2)], cbs[s], semi[s]).wait()

    def issue_g(s):
        pltpu.async_copy(v_hbm.at[sx[s]], vr[s], semg[s])

    def wait_g(s):
        pltpu.make_async_copy(v_hbm.at[sx[s]], vr[s], semg[s]).wait()

    for s0 in range(4):
        issue_idx(s0, s0)
    for s0 in range(3):
        wait_idx(s0, s0)
        issue_g(s0)

    def body4(t4, carry):
        for pp in range(4):
            t = t4 * 4 + pp
            s = pp
            p = pp % 2
            s3 = (pp + 3) % 4

            @pl.when(t + 3 < NB2)
            def _():
                wait_idx(t + 3, s3)
                issue_g(s3)

            wait_g(s)

            @pl.when(t >= 2)
            def _():
                pltpu.make_async_copy(mg[p], mspm.at[dsc[p]], sems[p]).wait()

            vrows = vr[s]
            msgb = mg[p]
            cb = cbs[s]
            dst_s = dsc[p]
            for i4 in range(BE2 // 16):
                dst_s[pl.ds(i4 * 16, 16)] = dx[s][pl.ds(i4 * 16, 16)]

            def jgroup(j, carry2):
                cv = cb[pl.ds(j * 16, 16)]
                for i in range(16):
                    e = j * 16 + i
                    ce = cv[i]
                    va, vb = plsc.unpack(vrows[e, pl.ds(off, 32)],
                                         format=plsc.PackFormat.INTERLEAVED)
                    msgb[e, 0:16] = ce * va
                    msgb[e, 16:32] = ce * vb
                return carry2

            lax.fori_loop(0, BE2 // 16, jgroup, 0)
            pltpu.async_copy(msgb, mspm.at[dst_s], sems[p], add=True)

            @pl.when(t + 4 < NB2)
            def _():
                issue_idx(t + 4, s)
        return carry

    lax.fori_loop(0, NB2 // 4, body4, 0)
    pltpu.make_async_copy(mg[0], mspm.at[dsc[0]], sems[0]).wait()
    pltpu.make_async_copy(mg[1], mspm.at[dsc[1]], sems[1]).wait()
    plsc.subcore_barrier()
    pltpu.sync_copy(mspm.at[pl.ds(sid * NROW_T, NROW_T)],
                    m_hbm.at[cid, pl.ds(sid * NROW_T, NROW_T)])


def _c_body(m_ref, xd_ref, h_ref, wdyn_ref, bdyn_ref,
            wir_m_ref, wiz_m_ref, win_m_ref, wir_d_ref, wiz_d_ref, win_d_ref,
            bir_ref, biz_ref, bin_ref,
            whr_ref, whz_ref, whn_ref, bhr_ref, bhz_ref, bhn_ref,
            gm_ref, btm_ref, gd_ref, btd_ref, gh_ref, bth_ref,
            out_ref):
    def ln(x, g, b):
        mu = jnp.mean(x, axis=-1, keepdims=True)
        var = jnp.mean((x - mu) ** 2, axis=-1, keepdims=True)
        return (x - mu) * jax.lax.rsqrt(var + 1e-5) * g + b

    m = ln(m_ref[...], gm_ref[...], btm_ref[...])
    d = ln(jnp.dot(xd_ref[...], wdyn_ref[...], preferred_element_type=_f32)
           + bdyn_ref[...], gd_ref[...], btd_ref[...])
    h = h_ref[...]

    def mm(x, w):
        return jnp.dot(x, w[...], preferred_element_type=_f32)

    i_r = mm(m, wir_m_ref) + mm(d, wir_d_ref) + bir_ref[...]
    i_z = mm(m, wiz_m_ref) + mm(d, wiz_d_ref) + biz_ref[...]
    i_n = mm(m, win_m_ref) + mm(d, win_d_ref) + bin_ref[...]
    h_r = mm(h, whr_ref) + bhr_ref[...]
    h_z = mm(h, whz_ref) + bhz_ref[...]
    h_n = mm(h, whn_ref) + bhn_ref[...]
    r = jax.nn.sigmoid(i_r + h_r)
    z = jax.nn.sigmoid(i_z + h_z)
    n = jnp.tanh(i_n + r * h_n)
    h_new = (1.0 - z) * n + z * h
    out_ref[...] = ln(h_new, gh_ref[...], bth_ref[...])


def kernel(h, x_static, x_dyn, edge_attr_static, edge_index,
           W_es1, b_es1, W_es2, b_es2, W_bw, b_bw, W_g1, b_g1, W_g2, b_g2,
           W_p1, b_p1, W_p2, b_p2, W_dyn, b_dyn, W_ih, b_ih, W_hh, b_hh,
           g_msg, bt_msg, g_dyn, bt_dyn, g_h, bt_h):
    r1 = lambda a: a.reshape(1, -1)

    gc = pl.pallas_call(
        _a0_body,
        out_shape=jax.ShapeDtypeStruct((8, 128), _f32),
    )(W_es2, W_bw, r1(b_es2), r1(b_bw), r1(b_g2))

    BN = 2000
    gridn = N // BN
    full = lambda shp: pl.BlockSpec(shp, lambda i: (0, 0))
    srct, dstt, vtab = pl.pallas_call(
        _a1_body,
        grid=(gridn,),
        in_specs=[
            pl.BlockSpec((BN, H), lambda i: (i, 0)),
            pl.BlockSpec((BN, DS), lambda i: (i, 0)),
            full((H, HID)), full((H, HID)), full((1, HID)),
            full((DS, HID)), full((DS, HID)),
            full((H, HID)), full((1, HID)), full((HID, MSG)), full((1, MSG)),
        ],
        out_specs=[
            pl.BlockSpec((BN, 256), lambda i: (i, 0)),
            pl.BlockSpec((BN, 256), lambda i: (i, 0)),
            pl.BlockSpec((BN, 128), lambda i: (i, 0)),
        ],
        out_shape=[
            jax.ShapeDtypeStruct((N, 256), _f32),
            jax.ShapeDtypeStruct((N, 256), _f32),
            jax.ShapeDtypeStruct((N, 128), jnp.bfloat16),
        ],
    )(h, x_static, W_g1[:H], W_g1[H:], r1(b_g1), W_es1[DE:DE + DS],
      W_es1[DE + DS:], W_p1, r1(b_p1), W_p2, r1(b_p2))

    ea_pad = jnp.pad(edge_attr_static, ((0, EPAD - E), (0, 0)))
    BEA = 8192
    ea_proj = pl.pallas_call(
        _a2_body,
        grid=(EPAD // BEA,),
        in_specs=[
            pl.BlockSpec((BEA, DE), lambda i: (i, 0)),
            full((DE, HID)), full((1, HID)),
        ],
        out_specs=pl.BlockSpec((BEA, HID), lambda i: (i, 0)),
        out_shape=jax.ShapeDtypeStruct((EPAD, HID), _f32),
    )(ea_pad, W_es1[:DE], r1(b_es1))

    src = jnp.pad(edge_index[0], (0, EPAD - E))
    dst = jnp.pad(edge_index[1], (0, EPAD - E))

    mesh = plsc.VectorSubcoreMesh(core_axis_name="c", subcore_axis_name="s")
    scp = pltpu.CompilerParams(needs_layout_passes=False,
                               use_tc_tiling_on_sc=False)
    c = pl.kernel(
        _p1_body,
        out_type=jax.ShapeDtypeStruct((EPAD,), _f32),
        mesh=mesh,
        compiler_params=scp,
        scratch_types=(
            [pltpu.VMEM((BE,), _i32) for _ in range(8)]
            + [pltpu.VMEM((BE, 256), _f32) for _ in range(8)]
            + [pltpu.VMEM((BE, 128), _f32) for _ in range(4)]
            + [pltpu.VMEM((BE,), _f32) for _ in range(2)]
            + [pltpu.VMEM((128,), _f32) for _ in range(3)]
            + [pltpu.VMEM((272,), _f32) for _ in range(2)]
            + [pltpu.SemaphoreType.DMA for _ in range(10)]
        ),
    )(src, dst, srct, dstt, ea_proj, gc, gc[0], W_g2[:, 0])

    zrows = jnp.zeros((NROW_T, 32), _f32)
    m2 = pl.kernel(
        _p2_body,
        out_type=jax.ShapeDtypeStruct((2, NPADM, 32), _f32),
        mesh=mesh,
        compiler_params=scp,
        scratch_types=(
            [pltpu.VMEM((BE2,), _i32) for _ in range(8)]
            + [pltpu.VMEM((BE2,), _f32) for _ in range(4)]
            + [pltpu.VMEM((BE2,), _i32) for _ in range(2)]
            + [pltpu.VMEM((BE2, 128), jnp.bfloat16) for _ in range(4)]
            + [pltpu.VMEM((BE2, 32), _f32) for _ in range(2)]
            + [pltpu.VMEM_SHARED((NPADM, 32), _f32)]
            + [pltpu.SemaphoreType.DMA for _ in range(10)]
        ),
    )(src, dst, c, vtab, zrows)

    inv = [(f // 2 if f % 2 == 0 else 16 + f // 2) for f in range(32)]
    m = jnp.concatenate([m2[0, :N][:, inv], m2[1, :N][:, inv]], axis=1)

    Wim, Wid = W_ih[:MSG], W_ih[MSG:]
    out = pl.pallas_call(
        _c_body,
        grid=(gridn,),
        in_specs=[
            pl.BlockSpec((BN, MSG), lambda i: (i, 0)),
            pl.BlockSpec((BN, DDYN), lambda i: (i, 0)),
            pl.BlockSpec((BN, H), lambda i: (i, 0)),
            full((DDYN, MSG)), full((1, MSG)),
            full((MSG, H)), full((MSG, H)), full((MSG, H)),
            full((MSG, H)), full((MSG, H)), full((MSG, H)),
            full((1, H)), full((1, H)), full((1, H)),
            full((H, H)), full((H, H)), full((H, H)),
            full((1, H)), full((1, H)), full((1, H)),
            full((1, MSG)), full((1, MSG)), full((1, MSG)), full((1, MSG)),
            full((1, H)), full((1, H)),
        ],
        out_specs=pl.BlockSpec((BN, H), lambda i: (i, 0)),
        out_shape=jax.ShapeDtypeStruct((N, H), _f32),
    )(m, x_dyn, h, W_dyn, r1(b_dyn),
      Wim[:, :H], Wim[:, H:2 * H], Wim[:, 2 * H:],
      Wid[:, :H], Wid[:, H:2 * H], Wid[:, 2 * H:],
      r1(b_ih[:H]), r1(b_ih[H:2 * H]), r1(b_ih[2 * H:]),
      W_hh[:, :H], W_hh[:, H:2 * H], W_hh[:, 2 * H:],
      r1(b_hh[:H]), r1(b_hh[H:2 * H]), r1(b_hh[2 * H:]),
      r1(g_msg), r1(bt_msg), r1(g_dyn), r1(bt_dyn),
      r1(g_h), r1(bt_h))
    return out


# speedup vs baseline: 1.1331x; 1.1061x over previous
import jax
import jax.numpy as jnp
from jax import lax
from jax.experimental import pallas as pl
from jax.experimental.pallas import tpu as pltpu
from jax.experimental.pallas import tpu_sc as plsc

N = 50000
E = 800000
H = 96
MSG = 64
HID = 128
DS = 16
DE = 16
DDYN = 8

NW = 32
EPAD = 819200
PER_W = EPAD // NW
BE = 64
NB1 = PER_W // BE
NJ1 = BE // 16
PER_T2 = EPAD // 16
BE2 = 64
NB2 = PER_T2 // BE2
NROW_T = 3128
NPADM = NROW_T * 16

_f32 = jnp.float32
_i32 = jnp.int32


def _a0_body(wes2_ref, wbw_ref, bes2_ref, bbw_ref, bg2_ref, gc_ref):
    w2row = lax.dot_general(
        wbw_ref[...], wes2_ref[...],
        dimension_numbers=(((0,), (1,)), ((), ())),
        preferred_element_type=_f32)
    c2 = jnp.dot(bes2_ref[...], wbw_ref[...],
                 preferred_element_type=_f32) + bbw_ref[...]
    io = lax.broadcasted_iota(_i32, (1, 128), 1)
    row1 = jnp.where(io == 0, c2[0, 0], 0.0) + jnp.where(io == 1, bg2_ref[0, 0], 0.0)
    gc_ref[0:1, :] = w2row
    gc_ref[1:2, :] = row1
    gc_ref[2:8, :] = jnp.zeros((6, 128), _f32)


def _a1_body(h_ref, xs_ref, wg1a_ref, wg1b_ref, bg1_ref, ws_ref, wd_ref,
             wp1_ref, bp1_ref, wp2_ref, bp2_ref,
             srct_ref, dstt_ref, v_ref):
    h = h_ref[...]
    xs = xs_ref[...]
    srct_ref[:, 0:128] = jnp.dot(xs, ws_ref[...], preferred_element_type=_f32)
    srct_ref[:, 128:256] = (jnp.dot(h, wg1a_ref[...], preferred_element_type=_f32)
                            + bg1_ref[...])
    dstt_ref[:, 0:128] = jnp.dot(xs, wd_ref[...], preferred_element_type=_f32)
    dstt_ref[:, 128:256] = jnp.dot(h, wg1b_ref[...], preferred_element_type=_f32)
    hp = jnp.maximum(jnp.dot(h, wp1_ref[...], preferred_element_type=_f32)
                     + bp1_ref[...], 0.0)
    v_ref[:, 0:MSG] = (jnp.dot(hp, wp2_ref[...], preferred_element_type=_f32)
                       + bp2_ref[...]).astype(jnp.bfloat16)
    v_ref[:, MSG:128] = jnp.zeros((h.shape[0], 128 - MSG), jnp.bfloat16)


def _a2_body(ea_ref, wea_ref, bes1_ref, out_ref):
    out_ref[...] = (jnp.dot(ea_ref[...], wea_ref[...], preferred_element_type=_f32)
                    + bes1_ref[...])


def _softplus16(x):
    a = jnp.abs(x)
    t = jnp.exp(-a)
    u = t / (2.0 + t)
    u2 = u * u
    at = u * (1.0 + u2 * (1.0 / 3.0 + u2 * (0.2 + u2 * (1.0 / 7.0 + u2 * (1.0 / 9.0)))))
    return jnp.maximum(x, 0.0) + 2.0 * at


def _p1_body(sidx_hbm, didx_hbm, srct_hbm, dstt_hbm, ea_hbm, gc_hbm,
             w2p_hbm, wg2p_hbm,
             c_hbm,
             sidx0, sidx1, didx0, didx1, srows0, srows1, drows0, drows1,
             ea0, ea1, cb0, cb1, w2v, wg2v, ccv, tb1, tb2,
             semi0, semi1, semg0, semg1, semo0, semo1):
    cid = lax.axis_index("c")
    sid = lax.axis_index("s")
    wid = sid * 2 + cid
    pltpu.sync_copy(gc_hbm.at[1], ccv)
    pltpu.sync_copy(w2p_hbm, w2v)
    pltpu.sync_copy(wg2p_hbm, wg2v)
    base0 = wid * PER_W
    sx = (sidx0, sidx1)
    dx = (didx0, didx1)
    sr = (srows0, srows1)
    dr = (drows0, drows1)
    eab = (ea0, ea1)
    cbs = (cb0, cb1)
    semi = (semi0, semi1)
    semg = (semg0, semg1)
    semo = (semo0, semo1)
    w2c = [w2v[pl.ds(16 * k, 16)] for k in range(8)]
    wg2c = [wg2v[pl.ds(16 * k, 16)] for k in range(8)]
    cc16 = ccv[0:16]
    c2 = cc16[0]
    bg2 = cc16[1]
    iota = lax.iota(_i32, 16)
    tcols = [iota * 17 + f for f in range(16)]

    def ibase(b):
        return base0 + b * BE

    def issue_idx(b, s):
        pltpu.async_copy(sidx_hbm.at[pl.ds(ibase(b), BE)], sx[s], semi[s])
        pltpu.async_copy(didx_hbm.at[pl.ds(ibase(b), BE)], dx[s], semi[s])

    def wait_idx(b, s):
        pltpu.make_async_copy(sidx_hbm.at[pl.ds(ibase(b), BE)], sx[s], semi[s]).wait()
        pltpu.make_async_copy(didx_hbm.at[pl.ds(ibase(b), BE)], dx[s], semi[s]).wait()

    def issue_g(b, s):
        pltpu.async_copy(srct_hbm.at[sx[s]], sr[s], semg[s])
        pltpu.async_copy(dstt_hbm.at[dx[s]], dr[s], semg[s])
        pltpu.async_copy(ea_hbm.at[pl.ds(ibase(b), BE)], eab[s], semg[s])

    def wait_g(b, s):
        pltpu.make_async_copy(srct_hbm.at[sx[s]], sr[s], semg[s]).wait()
        pltpu.make_async_copy(dstt_hbm.at[dx[s]], dr[s], semg[s]).wait()
        pltpu.make_async_copy(ea_hbm.at[pl.ds(ibase(b), BE)], eab[s], semg[s]).wait()

    IL = plsc.PackFormat.INTERLEAVED

    def compute_batch(base, s, p):
        srows = sr[s]
        drows = dr[s]
        earows = eab[s]
        cbuf = cbs[p]

        def jgroup(j, carry2):
            for i in range(16):
                e = j * 16 + i
                p1 = None
                p2 = None
                for k in range(8):
                    sl = pl.ds(16 * k, 16)
                    slg = pl.ds(128 + 16 * k, 16)
                    r1 = jnp.maximum(earows[e, sl] + srows[e, sl] + drows[e, sl], 0.0)
                    r2 = jnp.maximum(srows[e, slg] + drows[e, slg], 0.0)
                    q1 = r1 * w2c[k]
                    q2 = r2 * wg2c[k]
                    p1 = q1 if p1 is None else p1 + q1
                    p2 = q2 if p2 is None else p2 + q2
                tb1[pl.ds(i * 17, 16)] = p1
                tb2[pl.ds(i * 17, 16)] = p2
            u1 = None
            u2 = None
            for f in range(16):
                g1 = plsc.load_gather(tb1, [tcols[f]])
                g2 = plsc.load_gather(tb2, [tcols[f]])
                u1 = g1 if u1 is None else u1 + g1
                u2 = g2 if u2 is None else u2 + g2
            be = _softplus16(u1 + c2)
            ge = 1.0 / (1.0 + jnp.exp(-(u2 + bg2)))
            cv = be * ge
            eid = iota + (base + 16 * j)
            cv = jnp.where(eid < E, cv, 0.0)
            cbuf[pl.ds(16 * j, 16)] = cv
            return carry2

        lax.fori_loop(0, NJ1, jgroup, 0)

    issue_idx(0, 0)
    wait_idx(0, 0)
    issue_g(0, 0)
    issue_idx(1, 1)

    def body2(t2, carry):
        for pp in range(2):
            t = t2 * 2 + pp
            s = pp
            p = pp
            q = 1 - pp
            bn = t + 1

            @pl.when(bn < NB1)
            def _():
                wait_idx(bn, q)
                issue_g(bn, q)

            wait_g(t, s)

            @pl.when(t >= 2)
            def _():
                pltpu.make_async_copy(
                    cbs[p], c_hbm.at[pl.ds(ibase(t - 2), BE)], semo[p]).wait()

            compute_batch(ibase(t), s, p)
            pltpu.async_copy(cbs[p], c_hbm.at[pl.ds(ibase(t), BE)], semo[p])

            @pl.when(t + 2 < NB1)
            def _():
                issue_idx(t + 2, p)
        return carry

    lax.fori_loop(0, NB1 // 2, body2, 0)
    pltpu.make_async_copy(cbs[0], c_hbm.at[pl.ds(ibase(NB1 - 2), BE)], semo[0]).wait()
    pltpu.make_async_copy(cbs[1], c_hbm.at[pl.ds(ibase(NB1 - 1), BE)], semo[1]).wait()


def _p2_body(sidx_hbm, didx_hbm, c_hbm, v_hbm, z_hbm,
             m_hbm,
             sidx0, sidx1, sidx2, sidx3, didx0, didx1, didx2, didx3,
             cb0, cb1, cb2, cb3, dsc0, dsc1,
             vr0, vr1, vr2, vr3, mg0, mg1, mspm,
             semi0, semi1, semi2, semi3, semg0, semg1, semg2, semg3,
             sems0, sems1):
    cid = lax.axis_index("c")
    sid = lax.axis_index("s")
    pltpu.sync_copy(z_hbm, mspm.at[pl.ds(sid * NROW_T, NROW_T)])
    plsc.subcore_barrier()

    base0 = sid * PER_T2
    off = cid * 32
    sx = (sidx0, sidx1, sidx2, sidx3)
    dx = (didx0, didx1, didx2, didx3)
    cbs = (cb0, cb1, cb2, cb3)
    dsc = (dsc0, dsc1)
    vr = (vr0, vr1, vr2, vr3)
    mg = (mg0, mg1)
    semi = (semi0, semi1, semi2, semi3)
    semg = (semg0, semg1, semg2, semg3)
    sems = (sems0, sems1)

    def ibase(b):
        return base0 + b * BE2

    def issue_idx(b, s):
        pltpu.async_copy(sidx_hbm.at[pl.ds(ibase(b), BE2)], sx[s], semi[s])
        pltpu.async_copy(didx_hbm.at[pl.ds(ibase(b), BE2)], dx[s], semi[s])
        pltpu.async_copy(c_hbm.at[pl.ds(ibase(b), BE2)], cbs[s], semi[s])

    def wait_idx(b, s):
        pltpu.make_async_copy(sidx_hbm.at[pl.ds(ibase(b), BE2)], sx[s], semi[s]).wait()
        pltpu.make_async_copy(didx_hbm.at[pl.ds(ibase(b), BE2)], dx[s], semi[s]).wait()
        pltpu.make_async_copy(c_hbm.at[pl.ds(ibase(b), BE2)], cbs[s], semi[s]).wait()

    def issue_g(s):
        pltpu.async_copy(v_hbm.at[sx[s]], vr[s], semg[s])

    def wait_g(s):
        pltpu.make_async_copy(v_hbm.at[sx[s]], vr[s], semg[s]).wait()

    for s0 in range(4):
        issue_idx(s0, s0)
    for s0 in range(3):
        wait_idx(s0, s0)
        issue_g(s0)

    def body4(t4, carry):
        for pp in range(4):
            t = t4 * 4 + pp
            s = pp
            p = pp % 2
            s3 = (pp + 3) % 4

            @pl.when(t + 3 < NB2)
            def _():
                wait_idx(t + 3, s3)
                issue_g(s3)

            wait_g(s)

            @pl.when(t >= 2)
            def _():
                pltpu.make_async_copy(mg[p], mspm.at[dsc[p]], sems[p]).wait()

            vrows = vr[s]
            msgb = mg[p]
            cb = cbs[s]
            dst_s = dsc[p]
            for i4 in range(BE2 // 16):
                dst_s[pl.ds(i4 * 16, 16)] = dx[s][pl.ds(i4 * 16, 16)]

            def jgroup(j, carry2):
                cv = cb[pl.ds(j * 16, 16)]
                for i in range(16):
                    e = j * 16 + i
                    ce = cv[i]
                    va, vb = plsc.unpack(vrows[e, pl.ds(off, 32)],
                                         format=plsc.PackFormat.INTERLEAVED)
                    msgb[e, 0:16] = ce * va
                    msgb[e, 16:32] = ce * vb
                return carry2

            lax.fori_loop(0, BE2 // 16, jgroup, 0)
            pltpu.async_copy(msgb, mspm.at[dst_s], sems[p], add=True)

            @pl.when(t + 4 < NB2)
            def _():
                issue_idx(t + 4, s)
        return carry

    lax.fori_loop(0, NB2 // 4, body4, 0)
    pltpu.make_async_copy(mg[0], mspm.at[dsc[0]], sems[0]).wait()
    pltpu.make_async_copy(mg[1], mspm.at[dsc[1]], sems[1]).wait()
    plsc.subcore_barrier()
    pltpu.sync_copy(mspm.at[pl.ds(sid * NROW_T, NROW_T)],
                    m_hbm.at[cid, pl.ds(sid * NROW_T, NROW_T)])


def _c_body(m_ref, xd_ref, h_ref, wdyn_ref, bdyn_ref,
            wir_m_ref, wiz_m_ref, win_m_ref, wir_d_ref, wiz_d_ref, win_d_ref,
            bir_ref, biz_ref, bin_ref,
            whr_ref, whz_ref, whn_ref, bhr_ref, bhz_ref, bhn_ref,
            gm_ref, btm_ref, gd_ref, btd_ref, gh_ref, bth_ref,
            out_ref):
    def ln(x, g, b):
        mu = jnp.mean(x, axis=-1, keepdims=True)
        var = jnp.mean((x - mu) ** 2, axis=-1, keepdims=True)
        return (x - mu) * jax.lax.rsqrt(var + 1e-5) * g + b

    m = ln(m_ref[...], gm_ref[...], btm_ref[...])
    d = ln(jnp.dot(xd_ref[...], wdyn_ref[...], preferred_element_type=_f32)
           + bdyn_ref[...], gd_ref[...], btd_ref[...])
    h = h_ref[...]

    def mm(x, w):
        return jnp.dot(x, w[...], preferred_element_type=_f32)

    i_r = mm(m, wir_m_ref) + mm(d, wir_d_ref) + bir_ref[...]
    i_z = mm(m, wiz_m_ref) + mm(d, wiz_d_ref) + biz_ref[...]
    i_n = mm(m, win_m_ref) + mm(d, win_d_ref) + bin_ref[...]
    h_r = mm(h, whr_ref) + bhr_ref[...]
    h_z = mm(h, whz_ref) + bhz_ref[...]
    h_n = mm(h, whn_ref) + bhn_ref[...]
    r = jax.nn.sigmoid(i_r + h_r)
    z = jax.nn.sigmoid(i_z + h_z)
    n = jnp.tanh(i_n + r * h_n)
    h_new = (1.0 - z) * n + z * h
    out_ref[...] = ln(h_new, gh_ref[...], bth_ref[...])


def kernel(h, x_static, x_dyn, edge_attr_static, edge_index,
           W_es1, b_es1, W_es2, b_es2, W_bw, b_bw, W_g1, b_g1, W_g2, b_g2,
           W_p1, b_p1, W_p2, b_p2, W_dyn, b_dyn, W_ih, b_ih, W_hh, b_hh,
           g_msg, bt_msg, g_dyn, bt_dyn, g_h, bt_h):
    r1 = lambda a: a.reshape(1, -1)

    gc = pl.pallas_call(
        _a0_body,
        out_shape=jax.ShapeDtypeStruct((8, 128), _f32),
    )(W_es2, W_bw, r1(b_es2), r1(b_bw), r1(b_g2))

    BN = 2000
    gridn = N // BN
    full = lambda shp: pl.BlockSpec(shp, lambda i: (0, 0))
    srct, dstt, vtab = pl.pallas_call(
        _a1_body,
        grid=(gridn,),
        in_specs=[
            pl.BlockSpec((BN, H), lambda i: (i, 0)),
            pl.BlockSpec((BN, DS), lambda i: (i, 0)),
            full((H, HID)), full((H, HID)), full((1, HID)),
            full((DS, HID)), full((DS, HID)),
            full((H, HID)), full((1, HID)), full((HID, MSG)), full((1, MSG)),
        ],
        out_specs=[
            pl.BlockSpec((BN, 256), lambda i: (i, 0)),
            pl.BlockSpec((BN, 256), lambda i: (i, 0)),
            pl.BlockSpec((BN, 128), lambda i: (i, 0)),
        ],
        out_shape=[
            jax.ShapeDtypeStruct((N, 256), _f32),
            jax.ShapeDtypeStruct((N, 256), _f32),
            jax.ShapeDtypeStruct((N, 128), jnp.bfloat16),
        ],
    )(h, x_static, W_g1[:H], W_g1[H:], r1(b_g1), W_es1[DE:DE + DS],
      W_es1[DE + DS:], W_p1, r1(b_p1), W_p2, r1(b_p2))

    ea_pad = jnp.pad(edge_attr_static, ((0, EPAD - E), (0, 0)))
    BEA = 8192
    ea_proj = pl.pallas_call(
        _a2_body,
        grid=(EPAD // BEA,),
        in_specs=[
            pl.BlockSpec((BEA, DE), lambda i: (i, 0)),
            full((DE, HID)), full((1, HID)),
        ],
        out_specs=pl.BlockSpec((BEA, HID), lambda i: (i, 0)),
        out_shape=jax.ShapeDtypeStruct((EPAD, HID), _f32),
    )(ea_pad, W_es1[:DE], r1(b_es1))

    src = jnp.pad(edge_index[0], (0, EPAD - E))
    dst = jnp.pad(edge_index[1], (0, EPAD - E))

    mesh = plsc.VectorSubcoreMesh(core_axis_name="c", subcore_axis_name="s")
    scp = pltpu.CompilerParams(needs_layout_passes=False,
                               use_tc_tiling_on_sc=False)
    c = pl.kernel(
        _p1_body,
        out_type=jax.ShapeDtypeStruct((EPAD,), _f32),
        mesh=mesh,
        compiler_params=scp,
        scratch_types=(
            [pltpu.VMEM((BE,), _i32) for _ in range(4)]
            + [pltpu.VMEM((BE, 256), _f32) for _ in range(4)]
            + [pltpu.VMEM((BE, 128), _f32) for _ in range(2)]
            + [pltpu.VMEM((BE,), _f32) for _ in range(2)]
            + [pltpu.VMEM((128,), _f32) for _ in range(3)]
            + [pltpu.VMEM((272,), _f32) for _ in range(2)]
            + [pltpu.SemaphoreType.DMA for _ in range(6)]
        ),
    )(src, dst, srct, dstt, ea_proj, gc, gc[0], W_g2[:, 0])

    zrows = jnp.zeros((NROW_T, 32), _f32)
    m2 = pl.kernel(
        _p2_body,
        out_type=jax.ShapeDtypeStruct((2, NPADM, 32), _f32),
        mesh=mesh,
        compiler_params=scp,
        scratch_types=(
            [pltpu.VMEM((BE2,), _i32) for _ in range(8)]
            + [pltpu.VMEM((BE2,), _f32) for _ in range(4)]
            + [pltpu.VMEM((BE2,), _i32) for _ in range(2)]
            + [pltpu.VMEM((BE2, 128), jnp.bfloat16) for _ in range(4)]
            + [pltpu.VMEM((BE2, 32), _f32) for _ in range(2)]
            + [pltpu.VMEM_SHARED((NPADM, 32), _f32)]
            + [pltpu.SemaphoreType.DMA for _ in range(10)]
        ),
    )(src, dst, c, vtab, zrows)

    inv = [(f // 2 if f % 2 == 0 else 16 + f // 2) for f in range(32)]
    m = jnp.concatenate([m2[0, :N][:, inv], m2[1, :N][:, inv]], axis=1)

    Wim, Wid = W_ih[:MSG], W_ih[MSG:]
    out = pl.pallas_call(
        _c_body,
        grid=(gridn,),
        in_specs=[
            pl.BlockSpec((BN, MSG), lambda i: (i, 0)),
            pl.BlockSpec((BN, DDYN), lambda i: (i, 0)),
            pl.BlockSpec((BN, H), lambda i: (i, 0)),
            full((DDYN, MSG)), full((1, MSG)),
            full((MSG, H)), full((MSG, H)), full((MSG, H)),
            full((MSG, H)), full((MSG, H)), full((MSG, H)),
            full((1, H)), full((1, H)), full((1, H)),
            full((H, H)), full((H, H)), full((H, H)),
            full((1, H)), full((1, H)), full((1, H)),
            full((1, MSG)), full((1, MSG)), full((1, MSG)), full((1, MSG)),
            full((1, H)), full((1, H)),
        ],
        out_specs=pl.BlockSpec((BN, H), lambda i: (i, 0)),
        out_shape=jax.ShapeDtypeStruct((N, H), _f32),
    )(m, x_dyn, h, W_dyn, r1(b_dyn),
      Wim[:, :H], Wim[:, H:2 * H], Wim[:, 2 * H:],
      Wid[:, :H], Wid[:, H:2 * H], Wid[:, 2 * H:],
      r1(b_ih[:H]), r1(b_ih[H:2 * H]), r1(b_ih[2 * H:]),
      W_hh[:, :H], W_hh[:, H:2 * H], W_hh[:, 2 * H:],
      r1(b_hh[:H]), r1(b_hh[H:2 * H]), r1(b_hh[2 * H:]),
      r1(g_msg), r1(bt_msg), r1(g_dyn), r1(bt_dyn),
      r1(g_h), r1(bt_h))
    return out
